# baseline (speedup 1.0000x reference)
import jax
import jax.numpy as jnp
from jax import lax
from jax.experimental import pallas as pl
from jax.experimental.pallas import tpu as pltpu
from jax.experimental.pallas import tpu_sc as plsc

D = 32
B = 16384
H = 200
NC, NS, L = 2, 16, 16
NW = NC * NS
BPW = B // NW
CH = 16
NCH = BPW // CH
NB = 4
HP = 256


def _lane_bcast(vec, jvec):
    return lax.gather(
        vec, jvec,
        lax.GatherDimensionNumbers(offset_dims=(), collapsed_slice_dims=(0,),
                                   start_index_map=(0,)),
        (1,), mode=lax.GatherScatterMode.PROMISE_IN_BOUNDS)


def _pool_body(x1_hbm, x2_hbm, x3_hbm, table_hbm, p1_hbm, p2_hbm,
               x1s, x2s, x3s, rows1_v, rows2_v, out1_v, zeros_v, idxs_v,
               shp, sem_stage, sem_g, sem_sc):
    cid = lax.axis_index("c")
    sid = lax.axis_index("s")
    wid = sid * NC + cid
    base = wid * BPW

    jv = [jnp.full((L, 1), j, jnp.int32) for j in range(L)]
    zvec = jnp.zeros((L,), jnp.float32)

    def stage_issue(c, cb):
        b0 = base + c * CH
        pltpu.async_copy(x1_hbm.at[pl.ds(b0, CH)], x1s.at[cb], sem_stage.at[cb])
        pltpu.async_copy(x2_hbm.at[pl.ds(b0, CH)], x2s.at[cb], sem_stage.at[cb])
        pltpu.async_copy(x3_hbm.at[pl.ds(b0, CH)], x3s.at[cb], sem_stage.at[cb])

    def stage_wait(cb):
        pltpu.make_async_copy(x1_hbm.at[pl.ds(0, CH)], x1s.at[cb], sem_stage.at[cb]).wait()
        pltpu.make_async_copy(x2_hbm.at[pl.ds(0, CH)], x2s.at[cb], sem_stage.at[cb]).wait()
        pltpu.make_async_copy(x3_hbm.at[pl.ds(0, CH)], x3s.at[cb], sem_stage.at[cb]).wait()

    def gather_issue(r, p):
        c = r // CH
        cb = c % 2
        rr = r % CH
        pltpu.async_copy(table_hbm.at[x1s.at[cb, rr, pl.ds(0, 128)]],
                         rows1_v.at[p, pl.ds(0, 128)], sem_g.at[p])
        pltpu.async_copy(table_hbm.at[x1s.at[cb, rr, pl.ds(128, 72)]],
                         rows1_v.at[p, pl.ds(128, 72)], sem_g.at[p])
        pltpu.async_copy(table_hbm.at[x2s.at[cb, rr, pl.ds(0, 128)]],
                         rows2_v.at[p, pl.ds(0, 128)], sem_g.at[p])
        pltpu.async_copy(table_hbm.at[x2s.at[cb, rr, pl.ds(128, 72)]],
                         rows2_v.at[p, pl.ds(128, 72)], sem_g.at[p])

    def gather_wait(p):
        pltpu.make_async_copy(table_hbm.at[pl.ds(0, 128)],
                              rows1_v.at[p, pl.ds(0, 128)], sem_g.at[p]).wait()
        pltpu.make_async_copy(table_hbm.at[pl.ds(0, 72)],
                              rows1_v.at[p, pl.ds(128, 72)], sem_g.at[p]).wait()
        pltpu.make_async_copy(table_hbm.at[pl.ds(0, 128)],
                              rows2_v.at[p, pl.ds(0, 128)], sem_g.at[p]).wait()
        pltpu.make_async_copy(table_hbm.at[pl.ds(0, 72)],
                              rows2_v.at[p, pl.ds(128, 72)], sem_g.at[p]).wait()

    def scatter_issue(r, p):
        slot = sid * BPW + r
        sl = jnp.full((L,), slot, jnp.int32)
        for tr in range(2):
            for t in range(128 // L):
                idxs_v[p, tr, pl.ds(t * L, L)] = sl
        pltpu.async_copy(rows2_v.at[p, pl.ds(0, 128)], shp.at[idxs_v.at[p, 0]],
                         sem_sc.at[p], add=True)
        pltpu.async_copy(rows2_v.at[p, pl.ds(128, 128)], shp.at[idxs_v.at[p, 1]],
                         sem_sc.at[p], add=True)

    def scatter_wait(p):
        pltpu.make_async_copy(rows2_v.at[p, pl.ds(0, 128)],
                              shp.at[idxs_v.at[p, 0]], sem_sc.at[p]).wait()
        pltpu.make_async_copy(rows2_v.at[p, pl.ds(128, 128)],
                              shp.at[idxs_v.at[p, 1]], sem_sc.at[p]).wait()

    def zero_body(r2, carry):
        zeros_v[r2, pl.ds(0, L)] = zvec
        zeros_v[r2, pl.ds(L, L)] = zvec
        return carry
    lax.fori_loop(0, BPW, zero_body, 0)
    pltpu.sync_copy(zeros_v, shp.at[pl.ds(sid * BPW, BPW)])
    for q in range(NB):
        for i in range(H, HP):
            rows2_v[q, i, pl.ds(0, L)] = zvec
            rows2_v[q, i, pl.ds(L, L)] = zvec

    stage_issue(0, 0)
    stage_issue(1, 1)
    stage_wait(0)
    gather_issue(0, 0)
    gather_issue(1, 1)

    def row_body(r, carry):
        p = r % NB
        c = r // CH
        cb = c % 2
        rr = r % CH

        gather_wait(p)
        scatter_issue(r, p)

        @pl.when(jnp.logical_and(rr == CH - 2, r < BPW - 2))
        def _():
            stage_wait((c + 1) % 2)

        @pl.when(r < BPW - 2)
        def _():
            @pl.when(r >= 2)
            def _():
                scatter_wait((r + 2) % NB)
            gather_issue(r + 2, (r + 2) % NB)

        lo = [jnp.zeros((L,), jnp.float32) for _ in range(4)]
        hi = [jnp.zeros((L,), jnp.float32) for _ in range(4)]
        for ci in range(13):
            eb = 184 if ci == 12 else ci * 16
            w = x3s[cb, rr, pl.ds(eb, L)]
            for j in (range(8, 16) if ci == 12 else range(16)):
                i = eb + j
                k = j % 4
                wj = _lane_bcast(w, jv[j])
                lo[k] = lo[k] + wj * rows1_v[p, i, pl.ds(0, L)]
                hi[k] = hi[k] + wj * rows1_v[p, i, pl.ds(L, L)]
        out1_v[r, pl.ds(0, L)] = (lo[0] + lo[1]) + (lo[2] + lo[3])
        out1_v[r, pl.ds(L, L)] = (hi[0] + hi[1]) + (hi[2] + hi[3])

        @pl.when(jnp.logical_and(rr == CH - 1, c + 2 < NCH))
        def _():
            stage_issue(c + 2, cb)

        return carry

    lax.fori_loop(0, BPW, row_body, 0)
    for r in range(BPW - 4, BPW):
        scatter_wait(r % NB)
    pltpu.sync_copy(out1_v, p1_hbm.at[pl.ds(base, BPW)])
    pltpu.sync_copy(shp.at[pl.ds(sid * BPW, BPW)], p2_hbm.at[pl.ds(base, BPW)])


_pool = pl.kernel(
    _pool_body,
    out_type=(jax.ShapeDtypeStruct((B, D), jnp.float32),
              jax.ShapeDtypeStruct((B, D), jnp.float32)),
    mesh=plsc.VectorSubcoreMesh(core_axis_name="c", subcore_axis_name="s",
                                num_cores=NC, num_subcores=NS),
    scratch_types=[
        pltpu.VMEM((2, CH, H), jnp.int32),
        pltpu.VMEM((2, CH, H), jnp.int32),
        pltpu.VMEM((2, CH, H), jnp.float32),
        pltpu.VMEM((NB, H, D), jnp.float32),
        pltpu.VMEM((NB, HP, D), jnp.float32),
        pltpu.VMEM((BPW, D), jnp.float32),
        pltpu.VMEM((BPW, D), jnp.float32),
        pltpu.VMEM((NB, 2, 128), jnp.int32),
        pltpu.VMEM_SHARED((NS * BPW, D), jnp.float32),
        pltpu.SemaphoreType.DMA((2,)),
        pltpu.SemaphoreType.DMA((NB,)),
        pltpu.SemaphoreType.DMA((NB,)),
    ],
    compiler_params=pltpu.CompilerParams(use_tc_tiling_on_sc=False),
)


def _mlp_body(p1_ref, p2_ref, b1_ref, w1_ref, c1_ref, b2_ref, w2_ref, c2_ref,
              o_ref):
    v1 = jnp.tanh(p1_ref[...] + b1_ref[...])
    v1 = jnp.tanh(
        lax.dot_general(v1, w1_ref[...], (((1,), (1,)), ((), ())),
                        preferred_element_type=jnp.float32) + c1_ref[...])
    v2 = jnp.tanh(p2_ref[...] + b2_ref[...])
    v2 = jnp.tanh(
        lax.dot_general(v2, w2_ref[...], (((1,), (1,)), ((), ())),
                        preferred_element_type=jnp.float32) + c2_ref[...])
    o_ref[...] = jax.nn.sigmoid(jnp.sum(v1 * v2, axis=1))


_mlp = pl.pallas_call(
    _mlp_body,
    out_shape=jax.ShapeDtypeStruct((B,), jnp.float32),
)


@jax.jit
def kernel(x1, x2, x3, table, t1_bias1, t1_W, t1_b, t2_bias1, t2_W, t2_b):
    p1, p2 = _pool(x1, x2, x3, table)
    return _mlp(p1, p2, t1_bias1.reshape(1, D), t1_W, t1_b.reshape(1, D),
                t2_bias1.reshape(1, D), t2_W, t2_b.reshape(1, D))


# speedup vs baseline: 1.0865x; 1.0865x over previous
import jax
import jax.numpy as jnp
from jax import lax
from jax.experimental import pallas as pl
from jax.experimental.pallas import tpu as pltpu
from jax.experimental.pallas import tpu_sc as plsc

D = 32
B = 16384
H = 200
NC, NS, L = 2, 16, 16
NW = NC * NS
BPW = B // NW
CH = 16
NCH = BPW // CH
NB = 4
HP = 256


def _lane_bcast(vec, jvec):
    return lax.gather(
        vec, jvec,
        lax.GatherDimensionNumbers(offset_dims=(), collapsed_slice_dims=(0,),
                                   start_index_map=(0,)),
        (1,), mode=lax.GatherScatterMode.PROMISE_IN_BOUNDS)


def _pool_body(x1_hbm, x2_hbm, x3_hbm, table_hbm, p1_hbm, p2_hbm,
               x1s, x2s, x3s, rows1_v, rows2_v, out1_v, zeros_v, idxs_v,
               shp, sem_stage, sem_g, sem_sc):
    cid = lax.axis_index("c")
    sid = lax.axis_index("s")
    wid = sid * NC + cid
    base = wid * BPW

    jv = [jnp.full((L, 1), j, jnp.int32) for j in range(L)]
    zvec = jnp.zeros((L,), jnp.float32)

    def stage_issue(c, cb):
        b0 = base + c * CH
        pltpu.async_copy(x1_hbm.at[pl.ds(b0, CH)], x1s.at[cb], sem_stage.at[cb])
        pltpu.async_copy(x2_hbm.at[pl.ds(b0, CH)], x2s.at[cb], sem_stage.at[cb])
        pltpu.async_copy(x3_hbm.at[pl.ds(b0, CH)], x3s.at[cb], sem_stage.at[cb])

    def stage_wait(cb):
        pltpu.make_async_copy(x1_hbm.at[pl.ds(0, CH)], x1s.at[cb], sem_stage.at[cb]).wait()
        pltpu.make_async_copy(x2_hbm.at[pl.ds(0, CH)], x2s.at[cb], sem_stage.at[cb]).wait()
        pltpu.make_async_copy(x3_hbm.at[pl.ds(0, CH)], x3s.at[cb], sem_stage.at[cb]).wait()

    def gather_issue(r, p):
        c = r // CH
        cb = c % 2
        rr = r % CH
        pltpu.async_copy(table_hbm.at[x1s.at[cb, rr, pl.ds(0, 128)]],
                         rows1_v.at[p, pl.ds(0, 128)], sem_g.at[p])
        pltpu.async_copy(table_hbm.at[x1s.at[cb, rr, pl.ds(128, 72)]],
                         rows1_v.at[p, pl.ds(128, 72)], sem_g.at[p])
        pltpu.async_copy(table_hbm.at[x2s.at[cb, rr, pl.ds(0, 128)]],
                         rows2_v.at[p, pl.ds(0, 128)], sem_g.at[p])
        pltpu.async_copy(table_hbm.at[x2s.at[cb, rr, pl.ds(128, 72)]],
                         rows2_v.at[p, pl.ds(128, 72)], sem_g.at[p])

    def gather_wait(p):
        pltpu.make_async_copy(table_hbm.at[pl.ds(0, 128)],
                              rows1_v.at[p, pl.ds(0, 128)], sem_g.at[p]).wait()
        pltpu.make_async_copy(table_hbm.at[pl.ds(0, 72)],
                              rows1_v.at[p, pl.ds(128, 72)], sem_g.at[p]).wait()
        pltpu.make_async_copy(table_hbm.at[pl.ds(0, 128)],
                              rows2_v.at[p, pl.ds(0, 128)], sem_g.at[p]).wait()
        pltpu.make_async_copy(table_hbm.at[pl.ds(0, 72)],
                              rows2_v.at[p, pl.ds(128, 72)], sem_g.at[p]).wait()

    def scatter_issue(r, p):
        slot = sid * BPW + r
        sl = jnp.full((L,), slot, jnp.int32)
        for t in range(128 // L):
            idxs_v[p, pl.ds(t * L, L)] = sl
        pltpu.async_copy(rows2_v.at[p, pl.ds(0, 128)], shp.at[idxs_v.at[p]],
                         sem_sc.at[p], add=True)

    def scatter_wait(p):
        pltpu.make_async_copy(rows2_v.at[p, pl.ds(0, 128)],
                              shp.at[idxs_v.at[p]], sem_sc.at[p]).wait()

    def zero_body(r2, carry):
        zeros_v[r2, pl.ds(0, L)] = zvec
        zeros_v[r2, pl.ds(L, L)] = zvec
        return carry
    lax.fori_loop(0, BPW, zero_body, 0)
    pltpu.sync_copy(zeros_v, shp.at[pl.ds(sid * BPW, BPW)])

    stage_issue(0, 0)
    stage_issue(1, 1)
    stage_wait(0)
    gather_issue(0, 0)
    gather_issue(1, 1)

    def row_body(r, carry):
        p = r % NB
        c = r // CH
        cb = c % 2
        rr = r % CH

        gather_wait(p)
        scatter_issue(r, p)

        @pl.when(jnp.logical_and(rr == CH - 2, r < BPW - 2))
        def _():
            stage_wait((c + 1) % 2)

        @pl.when(r < BPW - 2)
        def _():
            @pl.when(r >= 2)
            def _():
                scatter_wait((r + 2) % NB)
            gather_issue(r + 2, (r + 2) % NB)

        lo = [jnp.zeros((L,), jnp.float32) for _ in range(4)]
        hi = [jnp.zeros((L,), jnp.float32) for _ in range(4)]
        b2lo = [jnp.zeros((L,), jnp.float32) for _ in range(2)]
        b2hi = [jnp.zeros((L,), jnp.float32) for _ in range(2)]
        for ci in range(13):
            eb = 184 if ci == 12 else ci * 16
            w = x3s[cb, rr, pl.ds(eb, L)]
            for j in (range(8, 16) if ci == 12 else range(16)):
                i = eb + j
                k = j % 4
                wj = _lane_bcast(w, jv[j])
                lo[k] = lo[k] + wj * rows1_v[p, i, pl.ds(0, L)]
                hi[k] = hi[k] + wj * rows1_v[p, i, pl.ds(L, L)]
                if i >= 128:
                    b2lo[j % 2] = b2lo[j % 2] + rows2_v[p, i, pl.ds(0, L)]
                    b2hi[j % 2] = b2hi[j % 2] + rows2_v[p, i, pl.ds(L, L)]
        out1_v[r, pl.ds(0, L)] = (lo[0] + lo[1]) + (lo[2] + lo[3])
        out1_v[r, pl.ds(L, L)] = (hi[0] + hi[1]) + (hi[2] + hi[3])
        zeros_v[r, pl.ds(0, L)] = b2lo[0] + b2lo[1]
        zeros_v[r, pl.ds(L, L)] = b2hi[0] + b2hi[1]

        @pl.when(jnp.logical_and(rr == CH - 1, c + 2 < NCH))
        def _():
            stage_issue(c + 2, cb)

        return carry

    lax.fori_loop(0, BPW, row_body, 0)
    for r in range(BPW - 4, BPW):
        scatter_wait(r % NB)
    pltpu.sync_copy(out1_v, p1_hbm.at[pl.ds(base, BPW)])
    pltpu.sync_copy(shp.at[pl.ds(sid * BPW, BPW)], out1_v)

    def merge_body(r2, carry):
        zeros_v[r2, pl.ds(0, L)] = zeros_v[r2, pl.ds(0, L)] + out1_v[r2, pl.ds(0, L)]
        zeros_v[r2, pl.ds(L, L)] = zeros_v[r2, pl.ds(L, L)] + out1_v[r2, pl.ds(L, L)]
        return carry
    lax.fori_loop(0, BPW, merge_body, 0)
    pltpu.sync_copy(zeros_v, p2_hbm.at[pl.ds(base, BPW)])


_pool = pl.kernel(
    _pool_body,
    out_type=(jax.ShapeDtypeStruct((B, D), jnp.float32),
              jax.ShapeDtypeStruct((B, D), jnp.float32)),
    mesh=plsc.VectorSubcoreMesh(core_axis_name="c", subcore_axis_name="s",
                                num_cores=NC, num_subcores=NS),
    scratch_types=[
        pltpu.VMEM((2, CH, H), jnp.int32),
        pltpu.VMEM((2, CH, H), jnp.int32),
        pltpu.VMEM((2, CH, H), jnp.float32),
        pltpu.VMEM((NB, H, D), jnp.float32),
        pltpu.VMEM((NB, H, D), jnp.float32),
        pltpu.VMEM((BPW, D), jnp.float32),
        pltpu.VMEM((BPW, D), jnp.float32),
        pltpu.VMEM((NB, 128), jnp.int32),
        pltpu.VMEM_SHARED((NS * BPW, D), jnp.float32),
        pltpu.SemaphoreType.DMA((2,)),
        pltpu.SemaphoreType.DMA((NB,)),
        pltpu.SemaphoreType.DMA((NB,)),
    ],
    compiler_params=pltpu.CompilerParams(use_tc_tiling_on_sc=False),
)


def _mlp_body(p1_ref, p2_ref, b1_ref, w1_ref, c1_ref, b2_ref, w2_ref, c2_ref,
              o_ref):
    v1 = jnp.tanh(p1_ref[...] + b1_ref[...])
    v1 = jnp.tanh(
        lax.dot_general(v1, w1_ref[...], (((1,), (1,)), ((), ())),
                        preferred_element_type=jnp.float32) + c1_ref[...])
    v2 = jnp.tanh(p2_ref[...] + b2_ref[...])
    v2 = jnp.tanh(
        lax.dot_general(v2, w2_ref[...], (((1,), (1,)), ((), ())),
                        preferred_element_type=jnp.float32) + c2_ref[...])
    o_ref[...] = jax.nn.sigmoid(jnp.sum(v1 * v2, axis=1))


_mlp = pl.pallas_call(
    _mlp_body,
    out_shape=jax.ShapeDtypeStruct((B,), jnp.float32),
)


@jax.jit
def kernel(x1, x2, x3, table, t1_bias1, t1_W, t1_b, t2_bias1, t2_W, t2_b):
    p1, p2 = _pool(x1, x2, x3, table)
    return _mlp(p1, p2, t1_bias1.reshape(1, D), t1_W, t1_b.reshape(1, D),
                t2_bias1.reshape(1, D), t2_W, t2_b.reshape(1, D))


# speedup vs baseline: 1.0951x; 1.0079x over previous
import jax
import jax.numpy as jnp
from jax import lax
from jax.experimental import pallas as pl
from jax.experimental.pallas import tpu as pltpu
from jax.experimental.pallas import tpu_sc as plsc

D = 32
B = 16384
H = 200
NC, NS, L = 2, 16, 16
NW = NC * NS
BPW = B // NW
CH = 16
NCH = BPW // CH
NB = 4
HP = 256


def _lane_bcast(vec, jvec):
    return lax.gather(
        vec, jvec,
        lax.GatherDimensionNumbers(offset_dims=(), collapsed_slice_dims=(0,),
                                   start_index_map=(0,)),
        (1,), mode=lax.GatherScatterMode.PROMISE_IN_BOUNDS)


def _pool_body(x1_hbm, x2_hbm, x3_hbm, table_hbm, p1_hbm, p2_hbm,
               x1s, x2s, x3s, rows1_v, rows2_v, out1_v, zeros_v, idxs_v,
               shp, sem_stage, sem_g, sem_sc):
    cid = lax.axis_index("c")
    sid = lax.axis_index("s")
    wid = sid * NC + cid
    base = wid * BPW

    jv = [jnp.full((L, 1), j, jnp.int32) for j in range(L)]
    zvec = jnp.zeros((L,), jnp.float32)

    def stage_issue(c, cb):
        b0 = base + c * CH
        pltpu.async_copy(x1_hbm.at[pl.ds(b0, CH)], x1s.at[cb], sem_stage.at[cb])
        pltpu.async_copy(x2_hbm.at[pl.ds(b0, CH)], x2s.at[cb], sem_stage.at[cb])
        pltpu.async_copy(x3_hbm.at[pl.ds(b0, CH)], x3s.at[cb], sem_stage.at[cb])

    def stage_wait(cb):
        pltpu.make_async_copy(x1_hbm.at[pl.ds(0, CH)], x1s.at[cb], sem_stage.at[cb]).wait()
        pltpu.make_async_copy(x2_hbm.at[pl.ds(0, CH)], x2s.at[cb], sem_stage.at[cb]).wait()
        pltpu.make_async_copy(x3_hbm.at[pl.ds(0, CH)], x3s.at[cb], sem_stage.at[cb]).wait()

    def gather_issue(r, p):
        c = r // CH
        cb = c % 2
        rr = r % CH
        pltpu.async_copy(table_hbm.at[x1s.at[cb, rr, pl.ds(0, 128)]],
                         rows1_v.at[p, pl.ds(0, 128)], sem_g.at[p])
        pltpu.async_copy(table_hbm.at[x1s.at[cb, rr, pl.ds(128, 72)]],
                         rows1_v.at[p, pl.ds(128, 72)], sem_g.at[p])
        pltpu.async_copy(table_hbm.at[x2s.at[cb, rr, pl.ds(0, 128)]],
                         rows2_v.at[p, pl.ds(0, 128)], sem_g.at[p])
        pltpu.async_copy(table_hbm.at[x2s.at[cb, rr, pl.ds(128, 72)]],
                         rows2_v.at[p, pl.ds(128, 72)], sem_g.at[p])

    def gather_wait(p):
        pltpu.make_async_copy(table_hbm.at[pl.ds(0, 128)],
                              rows1_v.at[p, pl.ds(0, 128)], sem_g.at[p]).wait()
        pltpu.make_async_copy(table_hbm.at[pl.ds(0, 72)],
                              rows1_v.at[p, pl.ds(128, 72)], sem_g.at[p]).wait()
        pltpu.make_async_copy(table_hbm.at[pl.ds(0, 128)],
                              rows2_v.at[p, pl.ds(0, 128)], sem_g.at[p]).wait()
        pltpu.make_async_copy(table_hbm.at[pl.ds(0, 72)],
                              rows2_v.at[p, pl.ds(128, 72)], sem_g.at[p]).wait()

    def scatter_issue(r, p):
        slot = sid * BPW + r
        sl = jnp.full((L,), slot, jnp.int32)
        for t in range(128 // L):
            idxs_v[p, pl.ds(t * L, L)] = sl
        pltpu.async_copy(rows2_v.at[p, pl.ds(0, 128)], shp.at[idxs_v.at[p]],
                         sem_sc.at[p], add=True)

    def scatter_wait(p):
        pltpu.make_async_copy(rows2_v.at[p, pl.ds(0, 128)],
                              shp.at[idxs_v.at[p]], sem_sc.at[p]).wait()

    def zero_body(r2, carry):
        zeros_v[r2, pl.ds(0, L)] = zvec
        zeros_v[r2, pl.ds(L, L)] = zvec
        return carry
    lax.fori_loop(0, BPW, zero_body, 0)
    pltpu.sync_copy(zeros_v, shp.at[pl.ds(sid * BPW, BPW)])

    stage_issue(0, 0)
    stage_issue(1, 1)
    stage_wait(0)
    gather_issue(0, 0)
    gather_issue(1, 1)
    gather_issue(2, 2)

    def row_body(r, carry):
        p = r % NB
        c = r // CH
        cb = c % 2
        rr = r % CH

        gather_wait(p)
        scatter_issue(r, p)

        @pl.when(jnp.logical_and(rr == CH - 3, r < BPW - 3))
        def _():
            stage_wait((c + 1) % 2)

        @pl.when(r < BPW - 3)
        def _():
            @pl.when(r >= 1)
            def _():
                scatter_wait((r + 3) % NB)
            gather_issue(r + 3, (r + 3) % NB)

        lo = [jnp.zeros((L,), jnp.float32) for _ in range(4)]
        hi = [jnp.zeros((L,), jnp.float32) for _ in range(4)]
        b2lo = [jnp.zeros((L,), jnp.float32) for _ in range(2)]
        b2hi = [jnp.zeros((L,), jnp.float32) for _ in range(2)]
        for ci in range(13):
            eb = 184 if ci == 12 else ci * 16
            w = x3s[cb, rr, pl.ds(eb, L)]
            for j in (range(8, 16) if ci == 12 else range(16)):
                i = eb + j
                k = j % 4
                wj = _lane_bcast(w, jv[j])
                lo[k] = lo[k] + wj * rows1_v[p, i, pl.ds(0, L)]
                hi[k] = hi[k] + wj * rows1_v[p, i, pl.ds(L, L)]
                if i >= 128:
                    b2lo[j % 2] = b2lo[j % 2] + rows2_v[p, i, pl.ds(0, L)]
                    b2hi[j % 2] = b2hi[j % 2] + rows2_v[p, i, pl.ds(L, L)]
        out1_v[r, pl.ds(0, L)] = (lo[0] + lo[1]) + (lo[2] + lo[3])
        out1_v[r, pl.ds(L, L)] = (hi[0] + hi[1]) + (hi[2] + hi[3])
        zeros_v[r, pl.ds(0, L)] = b2lo[0] + b2lo[1]
        zeros_v[r, pl.ds(L, L)] = b2hi[0] + b2hi[1]

        @pl.when(jnp.logical_and(rr == CH - 1, c + 2 < NCH))
        def _():
            stage_issue(c + 2, cb)

        return carry

    lax.fori_loop(0, BPW, row_body, 0)
    for r in range(BPW - 4, BPW):
        scatter_wait(r % NB)
    pltpu.sync_copy(out1_v, p1_hbm.at[pl.ds(base, BPW)])
    pltpu.sync_copy(shp.at[pl.ds(sid * BPW, BPW)], out1_v)

    def merge_body(r2, carry):
        zeros_v[r2, pl.ds(0, L)] = zeros_v[r2, pl.ds(0, L)] + out1_v[r2, pl.ds(0, L)]
        zeros_v[r2, pl.ds(L, L)] = zeros_v[r2, pl.ds(L, L)] + out1_v[r2, pl.ds(L, L)]
        return carry
    lax.fori_loop(0, BPW, merge_body, 0)
    pltpu.sync_copy(zeros_v, p2_hbm.at[pl.ds(base, BPW)])


_pool = pl.kernel(
    _pool_body,
    out_type=(jax.ShapeDtypeStruct((B, D), jnp.float32),
              jax.ShapeDtypeStruct((B, D), jnp.float32)),
    mesh=plsc.VectorSubcoreMesh(core_axis_name="c", subcore_axis_name="s",
                                num_cores=NC, num_subcores=NS),
    scratch_types=[
        pltpu.VMEM((2, CH, H), jnp.int32),
        pltpu.VMEM((2, CH, H), jnp.int32),
        pltpu.VMEM((2, CH, H), jnp.float32),
        pltpu.VMEM((NB, H, D), jnp.float32),
        pltpu.VMEM((NB, H, D), jnp.float32),
        pltpu.VMEM((BPW, D), jnp.float32),
        pltpu.VMEM((BPW, D), jnp.float32),
        pltpu.VMEM((NB, 128), jnp.int32),
        pltpu.VMEM_SHARED((NS * BPW, D), jnp.float32),
        pltpu.SemaphoreType.DMA((2,)),
        pltpu.SemaphoreType.DMA((NB,)),
        pltpu.SemaphoreType.DMA((NB,)),
    ],
    compiler_params=pltpu.CompilerParams(use_tc_tiling_on_sc=False),
)


def _mlp_body(p1_ref, p2_ref, b1_ref, w1_ref, c1_ref, b2_ref, w2_ref, c2_ref,
              o_ref):
    v1 = jnp.tanh(p1_ref[...] + b1_ref[...])
    v1 = jnp.tanh(
        lax.dot_general(v1, w1_ref[...], (((1,), (1,)), ((), ())),
                        preferred_element_type=jnp.float32) + c1_ref[...])
    v2 = jnp.tanh(p2_ref[...] + b2_ref[...])
    v2 = jnp.tanh(
        lax.dot_general(v2, w2_ref[...], (((1,), (1,)), ((), ())),
                        preferred_element_type=jnp.float32) + c2_ref[...])
    o_ref[...] = jax.nn.sigmoid(jnp.sum(v1 * v2, axis=1))


_mlp = pl.pallas_call(
    _mlp_body,
    out_shape=jax.ShapeDtypeStruct((B,), jnp.float32),
)


@jax.jit
def kernel(x1, x2, x3, table, t1_bias1, t1_W, t1_b, t2_bias1, t2_W, t2_b):
    p1, p2 = _pool(x1, x2, x3, table)
    return _mlp(p1, p2, t1_bias1.reshape(1, D), t1_W, t1_b.reshape(1, D),
                t2_bias1.reshape(1, D), t2_W, t2_b.reshape(1, D))


# speedup vs baseline: 1.0972x; 1.0020x over previous
import jax
import jax.numpy as jnp
from jax import lax
from jax.experimental import pallas as pl
from jax.experimental.pallas import tpu as pltpu
from jax.experimental.pallas import tpu_sc as plsc

D = 32
B = 16384
H = 200
NC, NS, L = 2, 16, 16
NW = NC * NS
BPW = B // NW
CH = 16
NCH = BPW // CH
NB = 4


def _lane_bcast(vec, jvec):
    return lax.gather(
        vec, jvec,
        lax.GatherDimensionNumbers(offset_dims=(), collapsed_slice_dims=(0,),
                                   start_index_map=(0,)),
        (1,), mode=lax.GatherScatterMode.PROMISE_IN_BOUNDS)


def _pool_body(x1_hbm, x2_hbm, x3_hbm, table_hbm, p1_hbm, p2_hbm,
               x1s, x2s, x3s, rows1_v, rows2_v, out1_v, zeros_v, idxs_v,
               shp, sem_stage, sem_g, sem_sc):
    cid = lax.axis_index("c")
    sid = lax.axis_index("s")
    wid = sid * NC + cid
    base = wid * BPW

    jv = [jnp.full((L, 1), j, jnp.int32) for j in range(L)]
    zvec = jnp.zeros((L,), jnp.float32)

    def stage_issue(c, cb):
        b0 = base + c * CH
        pltpu.async_copy(x1_hbm.at[pl.ds(b0, CH)], x1s.at[cb], sem_stage.at[cb])
        pltpu.async_copy(x2_hbm.at[pl.ds(b0, CH)], x2s.at[cb], sem_stage.at[cb])
        pltpu.async_copy(x3_hbm.at[pl.ds(b0, CH)], x3s.at[cb], sem_stage.at[cb])

    def stage_wait(cb):
        pltpu.make_async_copy(x1_hbm.at[pl.ds(0, CH)], x1s.at[cb], sem_stage.at[cb]).wait()
        pltpu.make_async_copy(x2_hbm.at[pl.ds(0, CH)], x2s.at[cb], sem_stage.at[cb]).wait()
        pltpu.make_async_copy(x3_hbm.at[pl.ds(0, CH)], x3s.at[cb], sem_stage.at[cb]).wait()

    def gather_issue(r, p):
        c = r // CH
        cb = c % 2
        rr = r % CH
        pltpu.async_copy(table_hbm.at[x1s.at[cb, rr, pl.ds(0, 128)]],
                         rows1_v.at[p, pl.ds(0, 128)], sem_g.at[p])
        pltpu.async_copy(table_hbm.at[x1s.at[cb, rr, pl.ds(128, 72)]],
                         rows1_v.at[p, pl.ds(128, 72)], sem_g.at[p])
        pltpu.async_copy(table_hbm.at[x2s.at[cb, rr, pl.ds(0, 128)]],
                         rows2_v.at[p, pl.ds(0, 128)], sem_g.at[p])
        pltpu.async_copy(table_hbm.at[x2s.at[cb, rr, pl.ds(128, 72)]],
                         rows2_v.at[p, pl.ds(128, 72)], sem_g.at[p])

    def gather_wait(p):
        pltpu.make_async_copy(table_hbm.at[pl.ds(0, 128)],
                              rows1_v.at[p, pl.ds(0, 128)], sem_g.at[p]).wait()
        pltpu.make_async_copy(table_hbm.at[pl.ds(0, 72)],
                              rows1_v.at[p, pl.ds(128, 72)], sem_g.at[p]).wait()
        pltpu.make_async_copy(table_hbm.at[pl.ds(0, 128)],
                              rows2_v.at[p, pl.ds(0, 128)], sem_g.at[p]).wait()
        pltpu.make_async_copy(table_hbm.at[pl.ds(0, 72)],
                              rows2_v.at[p, pl.ds(128, 72)], sem_g.at[p]).wait()

    def scatter_issue(r, p):
        slot = sid * BPW + r
        sl = jnp.full((L,), slot, jnp.int32)
        for t in range(128 // L):
            idxs_v[p, pl.ds(t * L, L)] = sl
        pltpu.async_copy(rows2_v.at[p, pl.ds(0, 128)], shp.at[idxs_v.at[p]],
                         sem_sc.at[p], add=True)

    def scatter_wait(p):
        pltpu.make_async_copy(rows2_v.at[p, pl.ds(0, 128)],
                              shp.at[idxs_v.at[p]], sem_sc.at[p]).wait()

    def zero_body(r2, carry):
        zeros_v[r2, pl.ds(0, L)] = zvec
        zeros_v[r2, pl.ds(L, L)] = zvec
        return carry
    lax.fori_loop(0, BPW, zero_body, 0)
    pltpu.sync_copy(zeros_v, shp.at[pl.ds(sid * BPW, BPW)])

    stage_issue(0, 0)
    stage_issue(1, 1)
    stage_wait(0)
    gather_issue(0, 0)
    gather_issue(1, 1)
    gather_issue(2, 2)

    def row_body(r, carry):
        p = r % NB
        c = r // CH
        cb = c % 2
        rr = r % CH

        gather_wait(p)
        scatter_issue(r, p)

        @pl.when(jnp.logical_and(rr == CH - 3, r < BPW - 3))
        def _():
            stage_wait((c + 1) % 2)

        @pl.when(r < BPW - 3)
        def _():
            @pl.when(r >= 1)
            def _():
                scatter_wait((r + 3) % NB)
            gather_issue(r + 3, (r + 3) % NB)

        lo = [jnp.zeros((L,), jnp.float32) for _ in range(4)]
        hi = [jnp.zeros((L,), jnp.float32) for _ in range(4)]
        b2lo = [jnp.zeros((L,), jnp.float32) for _ in range(2)]
        b2hi = [jnp.zeros((L,), jnp.float32) for _ in range(2)]
        for ci in range(13):
            eb = 184 if ci == 12 else ci * 16
            w = x3s[cb, rr, pl.ds(eb, L)]
            for j in (range(8, 16) if ci == 12 else range(16)):
                i = eb + j
                k = j % 4
                wj = _lane_bcast(w, jv[j])
                lo[k] = lo[k] + wj * rows1_v[p, i, pl.ds(0, L)]
                hi[k] = hi[k] + wj * rows1_v[p, i, pl.ds(L, L)]
                if i >= 128:
                    b2lo[j % 2] = b2lo[j % 2] + rows2_v[p, i, pl.ds(0, L)]
                    b2hi[j % 2] = b2hi[j % 2] + rows2_v[p, i, pl.ds(L, L)]
        out1_v[r, pl.ds(0, L)] = (lo[0] + lo[1]) + (lo[2] + lo[3])
        out1_v[r, pl.ds(L, L)] = (hi[0] + hi[1]) + (hi[2] + hi[3])
        zeros_v[r, pl.ds(0, L)] = b2lo[0] + b2lo[1]
        zeros_v[r, pl.ds(L, L)] = b2hi[0] + b2hi[1]

        @pl.when(jnp.logical_and(rr == CH - 1, c + 2 < NCH))
        def _():
            stage_issue(c + 2, cb)

        return carry

    lax.fori_loop(0, BPW, row_body, 0)
    for r in range(BPW - 4, BPW):
        scatter_wait(r % NB)
    pltpu.sync_copy(out1_v, p1_hbm.at[pl.ds(base, BPW)])
    pltpu.sync_copy(shp.at[pl.ds(sid * BPW, BPW)], out1_v)

    def merge_body(r2, carry):
        zeros_v[r2, pl.ds(0, L)] = zeros_v[r2, pl.ds(0, L)] + out1_v[r2, pl.ds(0, L)]
        zeros_v[r2, pl.ds(L, L)] = zeros_v[r2, pl.ds(L, L)] + out1_v[r2, pl.ds(L, L)]
        return carry
    lax.fori_loop(0, BPW, merge_body, 0)
    pltpu.sync_copy(zeros_v, p2_hbm.at[pl.ds(base, BPW)])


_pool = pl.kernel(
    _pool_body,
    out_type=(jax.ShapeDtypeStruct((B, D), jnp.float32),
              jax.ShapeDtypeStruct((B, D), jnp.float32)),
    mesh=plsc.VectorSubcoreMesh(core_axis_name="c", subcore_axis_name="s",
                                num_cores=NC, num_subcores=NS),
    scratch_types=[
        pltpu.VMEM((2, CH, H), jnp.int32),
        pltpu.VMEM((2, CH, H), jnp.int32),
        pltpu.VMEM((2, CH, H), jnp.float32),
        pltpu.VMEM((NB, H, D), jnp.float32),
        pltpu.VMEM((NB, H, D), jnp.float32),
        pltpu.VMEM((BPW, D), jnp.float32),
        pltpu.VMEM((BPW, D), jnp.float32),
        pltpu.VMEM((NB, 128), jnp.int32),
        pltpu.VMEM_SHARED((NS * BPW, D), jnp.float32),
        pltpu.SemaphoreType.DMA((2,)),
        pltpu.SemaphoreType.DMA((NB,)),
        pltpu.SemaphoreType.DMA((NB,)),
    ],
    compiler_params=pltpu.CompilerParams(use_tc_tiling_on_sc=False),
)


def _mlp_body(p1_ref, p2_ref, b1_ref, w1_ref, c1_ref, b2_ref, w2_ref, c2_ref,
              o_ref):
    v1 = jnp.tanh(p1_ref[...] + b1_ref[...])
    v1 = jnp.tanh(
        lax.dot_general(v1, w1_ref[...], (((1,), (1,)), ((), ())),
                        preferred_element_type=jnp.float32) + c1_ref[...])
    v2 = jnp.tanh(p2_ref[...] + b2_ref[...])
    v2 = jnp.tanh(
        lax.dot_general(v2, w2_ref[...], (((1,), (1,)), ((), ())),
                        preferred_element_type=jnp.float32) + c2_ref[...])
    o_ref[...] = jax.nn.sigmoid(jnp.sum(v1 * v2, axis=1))


_mlp = pl.pallas_call(
    _mlp_body,
    out_shape=jax.ShapeDtypeStruct((B,), jnp.float32),
)


@jax.jit
def kernel(x1, x2, x3, table, t1_bias1, t1_W, t1_b, t2_bias1, t2_W, t2_b):
    p1, p2 = _pool(x1, x2, x3, table)
    return _mlp(p1, p2, t1_bias1.reshape(1, D), t1_W, t1_b.reshape(1, D),
                t2_bias1.reshape(1, D), t2_W, t2_b.reshape(1, D))
